# Initial kernel scaffold; baseline (speedup 1.0000x reference)
import jax
import jax.numpy as jnp
from jax.experimental import pallas as pl


def kernel(src, etype, dst, segment_ids, w, coefficients, w0, b):
    raise NotImplementedError("write your pallas kernel here")


# speedup vs baseline: 3.4664x; 3.4664x over previous
import functools

import jax
import jax.numpy as jnp
from jax import lax
from jax.experimental import pallas as pl
from jax.experimental.pallas import tpu as pltpu
from jax.experimental.pallas import tpu_sc as plsc

_F32 = jnp.float32


def _message_body(et_ref, dst_ref, coef_ref, wcat_ref, out_ref,
                  *, be, nrel, nbasis, units, width):
    et = et_ref[0]
    oh = (et == lax.broadcasted_iota(jnp.int32, (be, nrel), 1)).astype(_F32)
    coef = jnp.dot(oh, coef_ref[...], preferred_element_type=_F32)
    h = jnp.dot(dst_ref[...], wcat_ref[...], preferred_element_type=_F32)
    x = coef[:, 0:1] * h[:, 0:units]
    for bq in range(1, nbasis):
        x = x + coef[:, bq:bq + 1] * h[:, bq * units:(bq + 1) * units]
    ones = jnp.ones((be, 1), _F32)
    zer = jnp.zeros((be, width - units - 1), _F32)
    out_ref[...] = jnp.concatenate([x, ones, zer], axis=1)


def _messages(et3, dst, coef_table, wcat, *, be, width):
    g = dst.shape[0] // be
    in_dim = dst.shape[1]
    nrel, nbasis = coef_table.shape
    units = wcat.shape[1] // nbasis
    body = functools.partial(_message_body, be=be, nrel=nrel, nbasis=nbasis,
                             units=units, width=width)
    return pl.pallas_call(
        body,
        grid=(g,),
        in_specs=[
            pl.BlockSpec((1, be, 1), lambda i: (i, 0, 0)),
            pl.BlockSpec((be, in_dim), lambda i: (i, 0)),
            pl.BlockSpec((nrel, nbasis), lambda i: (0, 0)),
            pl.BlockSpec((in_dim, nbasis * units), lambda i: (0, 0)),
        ],
        out_specs=pl.BlockSpec((be, width), lambda i: (i, 0)),
        out_shape=jax.ShapeDtypeStruct((dst.shape[0], width), _F32),
    )(et3, dst, coef_table, wcat)


def _segment_partials(xaug, seg2, n_pad, *, batch, nb):
    e, width = xaug.shape
    nc, ns = 2, 16
    nw = nc * ns
    epw = e // nw
    ch = batch * nb
    nchunk = epw // ch
    seg_rows_per_worker = epw // batch
    stripe = n_pad // ns
    mesh = plsc.VectorSubcoreMesh(core_axis_name="c", subcore_axis_name="s")

    def body(x_hbm, seg_hbm, out_hbm, rows_v, idx_v, zbuf, acc_sh):
        cid = lax.axis_index("c")
        sid = lax.axis_index("s")
        wid = sid * nc + cid

        def zrow(r, carry):
            for k in range(width // 16):
                zbuf[r, pl.ds(k * 16, 16)] = jnp.zeros((16,), _F32)
            return carry
        lax.fori_loop(0, stripe, zrow, None)
        pltpu.sync_copy(zbuf, acc_sh.at[pl.ds(sid * stripe, stripe), :])
        pltpu.sync_copy(seg_hbm.at[wid], idx_v)
        plsc.subcore_barrier()

        def chunk(c, carry):
            pltpu.sync_copy(x_hbm.at[pl.ds(wid * epw + c * ch, ch), :], rows_v)
            for bq in range(nb):
                pltpu.sync_copy(rows_v.at[pl.ds(bq * batch, batch), :],
                                acc_sh.at[idx_v.at[c * nb + bq]], add=True)
            return carry
        lax.fori_loop(0, nchunk, chunk, None)
        plsc.subcore_barrier()

        pltpu.sync_copy(acc_sh.at[pl.ds(sid * stripe, stripe), :],
                        out_hbm.at[cid, pl.ds(sid * stripe, stripe), :])

    fn = pl.kernel(
        body,
        out_type=jax.ShapeDtypeStruct((nc, n_pad, width), _F32),
        mesh=mesh,
        scratch_types=[
            pltpu.VMEM((ch, width), _F32),
            pltpu.VMEM((seg_rows_per_worker, batch), jnp.int32),
            pltpu.VMEM((stripe, width), _F32),
            pltpu.VMEM_SHARED((n_pad, width), _F32),
        ],
        compiler_params=pltpu.CompilerParams(use_tc_tiling_on_sc=False),
    )
    return fn(xaug, seg2)


def _final_body(p_ref, src_ref, w0_ref, b_ref, out_ref, *, n, units):
    p = p_ref[0, 0:n, :] + p_ref[1, 0:n, :]
    s = p[:, 0:units]
    cnt = p[:, units:units + 1]
    mean = jnp.where(cnt > 0, s / jnp.maximum(cnt, 1.0), 0.0)
    res = jnp.dot(src_ref[...], w0_ref[...], preferred_element_type=_F32)
    out_ref[...] = jnp.maximum(mean + res + b_ref[...], 0.0)


def _finalize(partial, src, w0, b2):
    n, units = src.shape[0], w0.shape[1]
    body = functools.partial(_final_body, n=n, units=units)
    return pl.pallas_call(
        body,
        out_shape=jax.ShapeDtypeStruct((n, units), _F32),
    )(partial, src, w0, b2)


def kernel(src, etype, dst, segment_ids, w, coefficients, w0, b):
    n, in_dim = src.shape
    e = dst.shape[0]
    nbasis, _, units = w.shape
    nrel = coefficients.shape[0]
    width = ((units + 1 + 15) // 16) * 16

    coef_table = coefficients[:, :, 0, 0]
    wcat = jnp.transpose(w, (1, 0, 2)).reshape(in_dim, nbasis * units)

    be = 3200
    et3 = etype.reshape(e // be, be, 1)
    xaug = _messages(et3, dst, coef_table, wcat, be=be, width=width)

    batch, nb = 80, 5
    seg2 = segment_ids.reshape(32, e // (32 * batch), batch)
    n_pad = ((n + 511) // 512) * 512
    partial = _segment_partials(xaug, seg2, n_pad, batch=batch, nb=nb)

    return _finalize(partial, src, w0, b.reshape(1, units))


# speedup vs baseline: 6.3499x; 1.8319x over previous
import functools

import jax
import jax.numpy as jnp
from jax import lax
from jax.experimental import pallas as pl
from jax.experimental.pallas import tpu as pltpu
from jax.experimental.pallas import tpu_sc as plsc

_F32 = jnp.float32


def _message_body(et_ref, dst_ref, cwide_ref, wcat_ref, fold_ref, out_ref,
                  *, be, nrel, units, width):
    et = et_ref[pl.ds(pl.program_id(0) * be, be)]
    oh = (lax.broadcasted_iota(jnp.int32, (nrel, be), 0)
          == et[None, :]).astype(_F32)
    coefwide = lax.dot_general(oh, cwide_ref[...], (((0,), (0,)), ((), ())),
                               preferred_element_type=_F32)
    h = jnp.dot(dst_ref[...], wcat_ref[...], preferred_element_type=_F32)
    xw = coefwide * h
    x48 = jnp.dot(xw, fold_ref[...], preferred_element_type=_F32)
    onescol = (lax.broadcasted_iota(jnp.int32, (1, width), 1)
               == units).astype(_F32)
    out_ref[...] = x48 + onescol


def _messages(etype, dst, cwide, wcat, fold, *, be, width):
    g = dst.shape[0] // be
    in_dim = dst.shape[1]
    nrel, bw = cwide.shape
    units = 32
    body = functools.partial(_message_body, be=be, nrel=nrel, units=units,
                             width=width)
    return pl.pallas_call(
        body,
        grid=(g,),
        in_specs=[
            pl.BlockSpec((dst.shape[0],), lambda i: (0,)),
            pl.BlockSpec((be, in_dim), lambda i: (i, 0)),
            pl.BlockSpec((nrel, bw), lambda i: (0, 0)),
            pl.BlockSpec((in_dim, bw), lambda i: (0, 0)),
            pl.BlockSpec((bw, width), lambda i: (0, 0)),
        ],
        out_specs=pl.BlockSpec((be, width), lambda i: (i, 0)),
        out_shape=jax.ShapeDtypeStruct((dst.shape[0], width), _F32),
    )(etype, dst, cwide, wcat, fold)


def _segment_partials(xaug, seg, n_pad, *, batch, nb):
    e, width = xaug.shape
    nc, ns = 2, 16
    nw = nc * ns
    epw = e // nw
    ch = batch * nb
    nchunk = epw // ch
    stripe = n_pad // ns
    mesh = plsc.VectorSubcoreMesh(core_axis_name="c", subcore_axis_name="s")

    def body(x_hbm, seg_hbm, out_hbm, rows_v, idx_v, zbuf, acc_sh):
        cid = lax.axis_index("c")
        sid = lax.axis_index("s")
        wid = sid * nc + cid

        def zrow(r, carry):
            for k in range(width // 16):
                zbuf[r, pl.ds(k * 16, 16)] = jnp.zeros((16,), _F32)
            return carry
        lax.fori_loop(0, stripe, zrow, None)
        pltpu.sync_copy(zbuf, acc_sh.at[pl.ds(sid * stripe, stripe), :])
        pltpu.sync_copy(seg_hbm.at[pl.ds(wid * epw, epw)], idx_v)
        plsc.subcore_barrier()

        def chunk(c, carry):
            pltpu.sync_copy(x_hbm.at[pl.ds(wid * epw + c * ch, ch), :], rows_v)
            for bq in range(nb):
                pltpu.sync_copy(
                    rows_v.at[pl.ds(bq * batch, batch), :],
                    acc_sh.at[idx_v.at[pl.ds((c * nb + bq) * batch, batch)]],
                    add=True)
            return carry
        lax.fori_loop(0, nchunk, chunk, None)
        plsc.subcore_barrier()

        pltpu.sync_copy(acc_sh.at[pl.ds(sid * stripe, stripe), :],
                        out_hbm.at[cid, pl.ds(sid * stripe, stripe), :])

    fn = pl.kernel(
        body,
        out_type=jax.ShapeDtypeStruct((nc, n_pad, width), _F32),
        mesh=mesh,
        scratch_types=[
            pltpu.VMEM((ch, width), _F32),
            pltpu.VMEM((epw,), jnp.int32),
            pltpu.VMEM((stripe, width), _F32),
            pltpu.VMEM_SHARED((n_pad, width), _F32),
        ],
        compiler_params=pltpu.CompilerParams(use_tc_tiling_on_sc=False),
    )
    return fn(xaug, seg)


def _final_body(p_ref, src_ref, w0_ref, b_ref, out_ref, *, n, units):
    p = p_ref[0, 0:n, :] + p_ref[1, 0:n, :]
    s = p[:, 0:units]
    cnt = p[:, units:units + 1]
    mean = jnp.where(cnt > 0, s / jnp.maximum(cnt, 1.0), 0.0)
    res = jnp.dot(src_ref[...], w0_ref[...], preferred_element_type=_F32)
    out_ref[...] = jnp.maximum(mean + res + b_ref[...], 0.0)


def _finalize(partial, src, w0, b2):
    n, units = src.shape[0], w0.shape[1]
    body = functools.partial(_final_body, n=n, units=units)
    return pl.pallas_call(
        body,
        out_shape=jax.ShapeDtypeStruct((n, units), _F32),
    )(partial, src, w0, b2)


def kernel(src, etype, dst, segment_ids, w, coefficients, w0, b):
    n, in_dim = src.shape
    e = dst.shape[0]
    nbasis, _, units = w.shape
    nrel = coefficients.shape[0]
    width = ((units + 1 + 15) // 16) * 16
    bw = nbasis * units

    coef_table = coefficients[:, :, 0, 0]
    cwide = jnp.repeat(coef_table, units, axis=1)
    wcat = jnp.transpose(w, (1, 0, 2)).reshape(in_dim, bw)
    fold = (jnp.arange(bw)[:, None] % units
            == jnp.arange(width)[None, :]).astype(_F32)

    be = 3200
    xaug = _messages(etype, dst, cwide, wcat, fold, be=be, width=width)

    batch, nb = 80, 5
    n_pad = ((n + 511) // 512) * 512
    partial = _segment_partials(xaug, segment_ids, n_pad, batch=batch, nb=nb)

    return _finalize(partial, src, w0, b.reshape(1, units))


# speedup vs baseline: 8.3525x; 1.3154x over previous
import functools

import jax
import jax.numpy as jnp
from jax import lax
from jax.experimental import pallas as pl
from jax.experimental.pallas import tpu as pltpu
from jax.experimental.pallas import tpu_sc as plsc

_F32 = jnp.float32


def _message_body(et_ref, dst_ref, cwide_ref, wcat_ref, fold_ref, out_ref,
                  *, be, nrel, units):
    et = et_ref[pl.ds(pl.program_id(0) * be, be)]
    oh = (lax.broadcasted_iota(jnp.int32, (nrel, be), 0)
          == et[None, :]).astype(_F32)
    coefwide = lax.dot_general(oh, cwide_ref[...], (((0,), (0,)), ((), ())),
                               preferred_element_type=_F32)
    h = jnp.dot(dst_ref[...], wcat_ref[...], preferred_element_type=_F32)
    xw = coefwide * h
    x = jnp.dot(xw, fold_ref[...], preferred_element_type=_F32)
    q = be // 4
    out_ref[...] = jnp.concatenate(
        [x[0:q], x[q:2 * q], x[2 * q:3 * q], x[3 * q:4 * q]], axis=1)


def _messages(etype, dst, cwide, wcat, fold, *, be):
    e = dst.shape[0]
    g = e // be
    in_dim = dst.shape[1]
    nrel, bw = cwide.shape
    units = fold.shape[1]
    body = functools.partial(_message_body, be=be, nrel=nrel, units=units)
    return pl.pallas_call(
        body,
        grid=(g,),
        in_specs=[
            pl.BlockSpec((e,), lambda i: (0,)),
            pl.BlockSpec((be, in_dim), lambda i: (i, 0)),
            pl.BlockSpec((nrel, bw), lambda i: (0, 0)),
            pl.BlockSpec((in_dim, bw), lambda i: (0, 0)),
            pl.BlockSpec((bw, units), lambda i: (0, 0)),
        ],
        out_specs=pl.BlockSpec((be // 4, 128), lambda i: (i, 0)),
        out_shape=jax.ShapeDtypeStruct((e // 4, 128), _F32),
    )(etype, dst, cwide, wcat, fold)


def _segment_partials(x32, seg, n_pad, *, batch, nb):
    e, units = x32.shape
    cw = 16
    nc, ns = 2, 16
    nw = nc * ns
    epw = e // nw
    ch = batch * nb
    nchunk = epw // ch
    stripe = n_pad // ns
    mesh = plsc.VectorSubcoreMesh(core_axis_name="c", subcore_axis_name="s")

    def body(x_hbm, seg_hbm, out_hbm, cnt_hbm,
             rows_v, idx_v, zbuf, ones_v, acc_sh, cnt_sh):
        cid = lax.axis_index("c")
        sid = lax.axis_index("s")
        wid = sid * nc + cid

        def zrow(r, carry):
            for k in range(units // 16):
                zbuf[r, pl.ds(k * 16, 16)] = jnp.zeros((16,), _F32)
            return carry
        lax.fori_loop(0, stripe, zrow, None)

        def orow(r, carry):
            ones_v[r, pl.ds(0, cw)] = jnp.ones((cw,), _F32)
            return carry
        lax.fori_loop(0, batch, orow, None)

        pltpu.sync_copy(zbuf, acc_sh.at[pl.ds(sid * stripe, stripe), :])
        pltpu.sync_copy(zbuf.at[:, pl.ds(0, cw)],
                        cnt_sh.at[pl.ds(sid * stripe, stripe), :])
        pltpu.sync_copy(seg_hbm.at[pl.ds(wid * epw, epw)], idx_v)
        plsc.subcore_barrier()

        def chunk(c, carry):
            pltpu.sync_copy(x_hbm.at[pl.ds(wid * epw + c * ch, ch), :], rows_v)
            for bq in range(nb):
                ids = idx_v.at[pl.ds((c * nb + bq) * batch, batch)]
                pltpu.sync_copy(rows_v.at[pl.ds(bq * batch, batch), :],
                                acc_sh.at[ids], add=True)
                pltpu.sync_copy(ones_v, cnt_sh.at[ids], add=True)
            return carry
        lax.fori_loop(0, nchunk, chunk, None)
        plsc.subcore_barrier()

        pltpu.sync_copy(acc_sh.at[pl.ds(sid * stripe, stripe), :],
                        out_hbm.at[cid, pl.ds(sid * stripe, stripe), :])
        pltpu.sync_copy(cnt_sh.at[pl.ds(sid * stripe, stripe), :],
                        cnt_hbm.at[cid, pl.ds(sid * stripe, stripe), :])

    fn = pl.kernel(
        body,
        out_type=(jax.ShapeDtypeStruct((nc, n_pad, units), _F32),
                  jax.ShapeDtypeStruct((nc, n_pad, cw), _F32)),
        mesh=mesh,
        scratch_types=[
            pltpu.VMEM((ch, units), _F32),
            pltpu.VMEM((epw,), jnp.int32),
            pltpu.VMEM((stripe, units), _F32),
            pltpu.VMEM((batch, cw), _F32),
            pltpu.VMEM_SHARED((n_pad, units), _F32),
            pltpu.VMEM_SHARED((n_pad, cw), _F32),
        ],
        compiler_params=pltpu.CompilerParams(use_tc_tiling_on_sc=False),
    )
    return fn(x32, seg)


def _final_body(p_ref, c_ref, src_ref, w0_ref, b_ref, out_ref, *, n):
    s = p_ref[0, 0:n, :] + p_ref[1, 0:n, :]
    cnt = c_ref[0, 0:n, 0:1] + c_ref[1, 0:n, 0:1]
    mean = jnp.where(cnt > 0, s / jnp.maximum(cnt, 1.0), 0.0)
    res = jnp.dot(src_ref[...], w0_ref[...], preferred_element_type=_F32)
    out_ref[...] = jnp.maximum(mean + res + b_ref[...], 0.0)


def _finalize(partial, counts, src, w0, b2):
    n, units = src.shape[0], w0.shape[1]
    body = functools.partial(_final_body, n=n)
    return pl.pallas_call(
        body,
        out_shape=jax.ShapeDtypeStruct((n, units), _F32),
    )(partial, counts, src, w0, b2)


def kernel(src, etype, dst, segment_ids, w, coefficients, w0, b):
    n, in_dim = src.shape
    e = dst.shape[0]
    nbasis, _, units = w.shape
    nrel = coefficients.shape[0]
    bw = nbasis * units

    coef_table = coefficients[:, :, 0, 0]
    cwide = jnp.repeat(coef_table, units, axis=1)
    wcat = jnp.transpose(w, (1, 0, 2)).reshape(in_dim, bw)
    fold = (jnp.arange(bw)[:, None] % units
            == jnp.arange(units)[None, :]).astype(_F32)

    be = 3200
    packed = _messages(etype, dst, cwide, wcat, fold, be=be)
    x32 = packed.reshape(e, units)
    seg_perm = segment_ids.reshape(e // be, 4, be // 4).swapaxes(1, 2).reshape(e)

    batch, nb = 80, 5
    n_pad = ((n + 511) // 512) * 512
    partial, counts = _segment_partials(x32, seg_perm, n_pad, batch=batch, nb=nb)

    return _finalize(partial, counts, src, w0, b.reshape(1, units))


# speedup vs baseline: 12.4919x; 1.4956x over previous
import functools

import jax
import jax.numpy as jnp
from jax import lax
from jax.experimental import pallas as pl
from jax.experimental.pallas import tpu as pltpu
from jax.experimental.pallas import tpu_sc as plsc

_F32 = jnp.float32


def _message_body(et_ref, d0_ref, d1_ref, d2_ref, d3_ref,
                  cwide_ref, wcat_ref, fold_ref, out_ref,
                  *, be, g4, nrel, units):
    i = pl.program_id(0)
    for aa, d_ref in enumerate((d0_ref, d1_ref, d2_ref, d3_ref)):
        et = et_ref[pl.ds((aa * g4 + i) * be, be)]
        oh = (lax.broadcasted_iota(jnp.int32, (nrel, be), 0)
              == et[None, :]).astype(jnp.bfloat16)
        coefwide = lax.dot_general(oh, cwide_ref[...],
                                   (((0,), (0,)), ((), ())),
                                   preferred_element_type=_F32)
        dstb = d_ref[...].astype(jnp.bfloat16)
        h = jnp.dot(dstb, wcat_ref[...], preferred_element_type=_F32)
        xw = (coefwide * h).astype(jnp.bfloat16)
        x = jnp.dot(xw, fold_ref[...], preferred_element_type=_F32)
        out_ref[:, aa * units:(aa + 1) * units] = x


def _messages(etype, dst, cwide, wcat, fold, *, be):
    e = dst.shape[0]
    g = e // be
    g4 = g // 4
    in_dim = dst.shape[1]
    nrel, bw = cwide.shape
    units = fold.shape[1]
    body = functools.partial(_message_body, be=be, g4=g4, nrel=nrel,
                             units=units)
    dspec = [pl.BlockSpec((be, in_dim), lambda i, aa=aa, g4=g4: (aa * g4 + i, 0))
             for aa in range(4)]
    return pl.pallas_call(
        body,
        grid=(g4,),
        in_specs=[pl.BlockSpec((e,), lambda i: (0,))] + dspec + [
            pl.BlockSpec((nrel, bw), lambda i: (0, 0)),
            pl.BlockSpec((in_dim, bw), lambda i: (0, 0)),
            pl.BlockSpec((bw, units), lambda i: (0, 0)),
        ],
        out_specs=pl.BlockSpec((be, 4 * units), lambda i: (i, 0)),
        out_shape=jax.ShapeDtypeStruct((e // 4, 4 * units), _F32),
    )(etype, dst, dst, dst, dst, cwide, wcat, fold)


def _segment_partials(packed, seg, n_pad, *, batch, nb):
    e4, lanes = packed.shape
    e = e4 * 4
    units = lanes // 4
    cw = 16
    nc, ns = 2, 16
    nw = nc * ns
    epw = e // nw
    ch = batch * nb
    nchunk = epw // ch
    stripe = n_pad // ns
    mesh = plsc.VectorSubcoreMesh(core_axis_name="c", subcore_axis_name="s")

    def body(x_hbm, seg_hbm, out_hbm, cnt_hbm,
             rows_v, idx_v, zbuf, ones_v, acc_sh, cnt_sh):
        cid = lax.axis_index("c")
        sid = lax.axis_index("s")
        wid = sid * nc + cid
        lane_a = wid // 8
        row_j = wid % 8

        def zrow(r, carry):
            for k in range(units // 16):
                zbuf[r, pl.ds(k * 16, 16)] = jnp.zeros((16,), _F32)
            return carry
        lax.fori_loop(0, stripe, zrow, None)

        def orow(r, carry):
            ones_v[r, pl.ds(0, cw)] = jnp.ones((cw,), _F32)
            return carry
        lax.fori_loop(0, batch, orow, None)

        pltpu.sync_copy(zbuf, acc_sh.at[pl.ds(sid * stripe, stripe), :])
        pltpu.sync_copy(zbuf.at[:, pl.ds(0, cw)],
                        cnt_sh.at[pl.ds(sid * stripe, stripe), :])
        pltpu.sync_copy(seg_hbm.at[pl.ds(wid * epw, epw)], idx_v)
        plsc.subcore_barrier()

        def chunk(c, carry):
            pltpu.sync_copy(
                x_hbm.at[pl.ds(row_j * epw + c * ch, ch),
                         pl.ds(lane_a * units, units)], rows_v)
            for bq in range(nb):
                ids = idx_v.at[pl.ds((c * nb + bq) * batch, batch)]
                pltpu.sync_copy(rows_v.at[pl.ds(bq * batch, batch), :],
                                acc_sh.at[ids], add=True)
                pltpu.sync_copy(ones_v, cnt_sh.at[ids], add=True)
            return carry
        lax.fori_loop(0, nchunk, chunk, None)
        plsc.subcore_barrier()

        pltpu.sync_copy(acc_sh.at[pl.ds(sid * stripe, stripe), :],
                        out_hbm.at[cid, pl.ds(sid * stripe, stripe), :])
        pltpu.sync_copy(cnt_sh.at[pl.ds(sid * stripe, stripe), :],
                        cnt_hbm.at[cid, pl.ds(sid * stripe, stripe), :])

    fn = pl.kernel(
        body,
        out_type=(jax.ShapeDtypeStruct((nc, n_pad, units), _F32),
                  jax.ShapeDtypeStruct((nc, n_pad, cw), _F32)),
        mesh=mesh,
        scratch_types=[
            pltpu.VMEM((ch, units), _F32),
            pltpu.VMEM((epw,), jnp.int32),
            pltpu.VMEM((stripe, units), _F32),
            pltpu.VMEM((batch, cw), _F32),
            pltpu.VMEM_SHARED((n_pad, units), _F32),
            pltpu.VMEM_SHARED((n_pad, cw), _F32),
        ],
        compiler_params=pltpu.CompilerParams(use_tc_tiling_on_sc=False),
    )
    return fn(packed, seg)


def _final_body(p_ref, c_ref, src_ref, w0_ref, b_ref, out_ref, *, n):
    s = p_ref[0, 0:n, :] + p_ref[1, 0:n, :]
    cnt = c_ref[0, 0:n, 0:1] + c_ref[1, 0:n, 0:1]
    mean = jnp.where(cnt > 0, s / jnp.maximum(cnt, 1.0), 0.0)
    res = jnp.dot(src_ref[...], w0_ref[...], preferred_element_type=_F32)
    out_ref[...] = jnp.maximum(mean + res + b_ref[...], 0.0)


def _finalize(partial, counts, src, w0, b2):
    n, units = src.shape[0], w0.shape[1]
    body = functools.partial(_final_body, n=n)
    return pl.pallas_call(
        body,
        out_shape=jax.ShapeDtypeStruct((n, units), _F32),
    )(partial, counts, src, w0, b2)


def kernel(src, etype, dst, segment_ids, w, coefficients, w0, b):
    n, in_dim = src.shape
    e = dst.shape[0]
    nbasis, _, units = w.shape
    nrel = coefficients.shape[0]
    bw = nbasis * units

    coef_table = coefficients[:, :, 0, 0]
    cwide = jnp.repeat(coef_table, units, axis=1).astype(jnp.bfloat16)
    wcat = jnp.transpose(w, (1, 0, 2)).reshape(in_dim, bw).astype(jnp.bfloat16)
    fold = (jnp.arange(bw)[:, None] % units
            == jnp.arange(units)[None, :]).astype(jnp.bfloat16)

    be = 3200
    packed = _messages(etype, dst, cwide, wcat, fold, be=be)

    batch, nb = 80, 5
    n_pad = ((n + 511) // 512) * 512
    partial, counts = _segment_partials(packed, segment_ids, n_pad,
                                        batch=batch, nb=nb)

    return _finalize(partial, counts, src, w0, b.reshape(1, units))


# speedup vs baseline: 15.0294x; 1.2031x over previous
import functools

import jax
import jax.numpy as jnp
from jax import lax
from jax.experimental import pallas as pl
from jax.experimental.pallas import tpu as pltpu
from jax.experimental.pallas import tpu_sc as plsc

_F32 = jnp.float32
_BF16 = jnp.bfloat16


def _message_body(et_ref, d4_ref, cwide_ref, wcat_ref, fold_ref, out_ref,
                  *, be, g4, nrel, units):
    i = pl.program_id(0)
    ets = [et_ref[pl.ds((aa * g4 + i) * be, be)][None, :] for aa in range(4)]
    et_all = jnp.concatenate(ets, axis=1)
    oh = (lax.broadcasted_iota(jnp.int32, (nrel, 4 * be), 0)
          == et_all).astype(_BF16)
    coefwide = lax.dot_general(oh, cwide_ref[...], (((0,), (0,)), ((), ())),
                               preferred_element_type=_F32)
    d = jnp.reshape(d4_ref[...], (4 * be, d4_ref.shape[-1])).astype(_BF16)
    h = jnp.dot(d, wcat_ref[...], preferred_element_type=_F32)
    xw = (coefwide * h).astype(_BF16)
    x = jnp.dot(xw, fold_ref[...], preferred_element_type=_F32)
    out_ref[...] = jnp.concatenate(
        [x[aa * be:(aa + 1) * be] for aa in range(4)], axis=1)


def _messages(etype, dst, cwide, wcat, fold, *, be):
    e, in_dim = dst.shape
    g = e // be
    g4 = g // 4
    nrel, bw = cwide.shape
    units = fold.shape[1]
    dst4 = dst.reshape(4, g4, be, in_dim)
    body = functools.partial(_message_body, be=be, g4=g4, nrel=nrel,
                             units=units)
    return pl.pallas_call(
        body,
        grid=(g4,),
        in_specs=[
            pl.BlockSpec((e,), lambda i: (0,)),
            pl.BlockSpec((4, 1, be, in_dim), lambda i: (0, i, 0, 0)),
            pl.BlockSpec((nrel, bw), lambda i: (0, 0)),
            pl.BlockSpec((in_dim, bw), lambda i: (0, 0)),
            pl.BlockSpec((bw, units), lambda i: (0, 0)),
        ],
        out_specs=pl.BlockSpec((be, 4 * units), lambda i: (i, 0)),
        out_shape=jax.ShapeDtypeStruct((e // 4, 4 * units), _F32),
    )(etype, dst4, cwide, wcat, fold)


def _segment_partials(packed, seg, n_pad, *, batch, nb):
    e4, lanes = packed.shape
    e = e4 * 4
    units = lanes // 4
    nc, ns = 2, 16
    nw = nc * ns
    epw = e // nw
    ch = batch * nb
    nchunk = epw // ch
    stripe = n_pad // ns
    mesh = plsc.VectorSubcoreMesh(core_axis_name="c", subcore_axis_name="s")

    def body(x_hbm, seg_hbm, out_hbm, cnt_hbm,
             rows0, rows1, idx_v, zbuf, ones_v, acc_sh, cnt_sh, sem_g, sem_s):
        cid = lax.axis_index("c")
        sid = lax.axis_index("s")
        wid = sid * nc + cid
        lane_a = wid // 8
        row_j = wid % 8

        def zrow(r, carry):
            for k in range(units // 16):
                zbuf[r, pl.ds(k * 16, 16)] = jnp.zeros((16,), _F32)
            return carry
        lax.fori_loop(0, stripe, zrow, None)

        def orow(r, carry):
            for k in range(units // 16):
                ones_v[r, pl.ds(k * 16, 16)] = jnp.ones((16,), _F32)
            return carry
        lax.fori_loop(0, batch, orow, None)

        pltpu.sync_copy(zbuf, acc_sh.at[pl.ds(sid * stripe, stripe), :])
        pltpu.sync_copy(zbuf, cnt_sh.at[pl.ds(sid * stripe, stripe), :])
        pltpu.sync_copy(seg_hbm.at[pl.ds(wid * epw, epw)], idx_v)
        plsc.subcore_barrier()

        def xsrc(c):
            return x_hbm.at[pl.ds(row_j * epw + c * ch, ch),
                            pl.ds(lane_a * units, units)]

        rows = (rows0, rows1)
        pltpu.async_copy(xsrc(0), rows0, sem_g)

        def chunk(c, carry):
            for jj in range(2):
                @pl.when(c % 2 == jj)
                def _():
                    buf = rows[jj]
                    pltpu.make_async_copy(xsrc(c), buf, sem_g).wait()

                    @pl.when(c + 1 < nchunk)
                    def _():
                        pltpu.async_copy(xsrc(c + 1), rows[1 - jj], sem_g)

                    descs = []
                    for bq in range(nb):
                        ids = idx_v.at[pl.ds((c * nb + bq) * batch, batch)]
                        descs.append(pltpu.async_copy(
                            buf.at[pl.ds(bq * batch, batch), :],
                            acc_sh.at[ids], sem_s, add=True))
                        descs.append(pltpu.async_copy(
                            ones_v, cnt_sh.at[ids], sem_s, add=True))
                    for dsc in descs:
                        dsc.wait()
            return carry
        lax.fori_loop(0, nchunk, chunk, None)
        plsc.subcore_barrier()

        pltpu.sync_copy(acc_sh.at[pl.ds(sid * stripe, stripe), :],
                        out_hbm.at[cid, pl.ds(sid * stripe, stripe), :])
        pltpu.sync_copy(cnt_sh.at[pl.ds(sid * stripe, stripe), :],
                        cnt_hbm.at[cid, pl.ds(sid * stripe, stripe), :])

    fn = pl.kernel(
        body,
        out_type=(jax.ShapeDtypeStruct((nc, n_pad, units), _F32),
                  jax.ShapeDtypeStruct((nc, n_pad, units), _F32)),
        mesh=mesh,
        scratch_types=[
            pltpu.VMEM((ch, units), _F32),
            pltpu.VMEM((ch, units), _F32),
            pltpu.VMEM((epw,), jnp.int32),
            pltpu.VMEM((stripe, units), _F32),
            pltpu.VMEM((batch, units), _F32),
            pltpu.VMEM_SHARED((n_pad, units), _F32),
            pltpu.VMEM_SHARED((n_pad, units), _F32),
            pltpu.SemaphoreType.DMA,
            pltpu.SemaphoreType.DMA,
        ],
        compiler_params=pltpu.CompilerParams(use_tc_tiling_on_sc=False),
    )
    return fn(packed, seg)


def _final_body(p_ref, c_ref, srcq_ref, w4_ref, b4_ref, out_ref, *, nq):
    sp = p_ref[0] + p_ref[1]
    cp = c_ref[0] + c_ref[1]
    mean = jnp.where(cp > 0, sp / jnp.maximum(cp, 1.0), 0.0)[0:nq]
    res = jnp.dot(srcq_ref[...], w4_ref[...], preferred_element_type=_F32)
    out_ref[...] = jnp.maximum(mean + res + b4_ref[...], 0.0)


def _finalize(partial_p, counts_p, srcq, w4, b4):
    nq = srcq.shape[0]
    body = functools.partial(_final_body, nq=nq)
    return pl.pallas_call(
        body,
        out_shape=jax.ShapeDtypeStruct((nq, w4.shape[1]), _F32),
    )(partial_p, counts_p, srcq, w4, b4)


def kernel(src, etype, dst, segment_ids, w, coefficients, w0, b):
    n, in_dim = src.shape
    e = dst.shape[0]
    nbasis, _, units = w.shape
    nrel = coefficients.shape[0]
    bw = nbasis * units

    coef_table = coefficients[:, :, 0, 0]
    cwide = jnp.repeat(coef_table, units, axis=1).astype(_BF16)
    wcat = jnp.transpose(w, (1, 0, 2)).reshape(in_dim, bw).astype(_BF16)
    fold = (jnp.arange(bw)[:, None] % units
            == jnp.arange(units)[None, :]).astype(_BF16)

    be = 3200
    packed = _messages(etype, dst, cwide, wcat, fold, be=be)

    batch, nb = 80, 5
    n_pad = ((n + 511) // 512) * 512
    partial, counts = _segment_partials(packed, segment_ids, n_pad,
                                        batch=batch, nb=nb)

    partial_p = partial.reshape(2, n_pad // 4, 4 * units)
    counts_p = counts.reshape(2, n_pad // 4, 4 * units)
    srcq = src.reshape(n // 4, 4 * in_dim)
    w4 = jnp.kron(jnp.eye(4, dtype=_F32), w0)
    b4 = jnp.tile(b, 4).reshape(1, 4 * units)
    out_p = _finalize(partial_p, counts_p, srcq, w4, b4)
    return out_p.reshape(n, units)


# speedup vs baseline: 15.0394x; 1.0007x over previous
import functools

import jax
import jax.numpy as jnp
from jax import lax
from jax.experimental import pallas as pl
from jax.experimental.pallas import tpu as pltpu
from jax.experimental.pallas import tpu_sc as plsc

_F32 = jnp.float32
_BF16 = jnp.bfloat16


def _message_body(et_ref, d4_ref, cwide_ref, wcat_ref, fold_ref, out_ref,
                  *, be, g4, nrel, units):
    i = pl.program_id(0)
    ets = [et_ref[pl.ds((aa * g4 + i) * be, be)][None, :] for aa in range(4)]
    et_all = jnp.concatenate(ets, axis=1)
    oh = (lax.broadcasted_iota(jnp.int32, (nrel, 4 * be), 0)
          == et_all).astype(_BF16)
    coefwide = lax.dot_general(oh, cwide_ref[...], (((0,), (0,)), ((), ())),
                               preferred_element_type=_F32)
    d = jnp.reshape(d4_ref[...], (4 * be, d4_ref.shape[-1])).astype(_BF16)
    h = jnp.dot(d, wcat_ref[...], preferred_element_type=_F32)
    xw = (coefwide * h).astype(_BF16)
    x = jnp.dot(xw, fold_ref[...], preferred_element_type=_F32)
    out_ref[...] = jnp.concatenate(
        [x[aa * be:(aa + 1) * be] for aa in range(4)], axis=1)


def _messages(etype, dst, cwide, wcat, fold, *, be):
    e, in_dim = dst.shape
    g = e // be
    g4 = g // 4
    nrel, bw = cwide.shape
    units = fold.shape[1]
    dst4 = dst.reshape(4, g4, be, in_dim)
    body = functools.partial(_message_body, be=be, g4=g4, nrel=nrel,
                             units=units)
    return pl.pallas_call(
        body,
        grid=(g4,),
        in_specs=[
            pl.BlockSpec((e,), lambda i: (0,)),
            pl.BlockSpec((4, 1, be, in_dim), lambda i: (0, i, 0, 0)),
            pl.BlockSpec((nrel, bw), lambda i: (0, 0)),
            pl.BlockSpec((in_dim, bw), lambda i: (0, 0)),
            pl.BlockSpec((bw, units), lambda i: (0, 0)),
        ],
        out_specs=pl.BlockSpec((be, 4 * units), lambda i: (i, 0)),
        out_shape=jax.ShapeDtypeStruct((e // 4, 4 * units), _F32),
    )(etype, dst4, cwide, wcat, fold)


def _segment_partials(packed, seg, n_pad, *, batch, nb):
    e4, lanes = packed.shape
    e = e4 * 4
    units = lanes // 4
    nc, ns = 2, 16
    nw = nc * ns
    epw = e // nw
    ch = batch * nb
    nchunk = epw // ch
    stripe = n_pad // ns
    mesh = plsc.VectorSubcoreMesh(core_axis_name="c", subcore_axis_name="s")

    def body(x_hbm, seg_hbm, out_hbm, cnt_hbm,
             rows0, rows1, idx_v, zbuf, ones_v, acc_sh, cnt_sh, sem_g, sem_s):
        cid = lax.axis_index("c")
        sid = lax.axis_index("s")
        wid = sid * nc + cid
        lane_a = wid // 8
        row_j = wid % 8

        def zrow(r, carry):
            for k in range(units // 16):
                zbuf[r, pl.ds(k * 16, 16)] = jnp.zeros((16,), _F32)
            return carry
        lax.fori_loop(0, stripe, zrow, None)

        def orow(r, carry):
            for k in range(units // 16):
                ones_v[r, pl.ds(k * 16, 16)] = jnp.ones((16,), _F32)
            return carry
        lax.fori_loop(0, batch, orow, None)

        pltpu.sync_copy(zbuf, acc_sh.at[pl.ds(sid * stripe, stripe), :])
        pltpu.sync_copy(zbuf, cnt_sh.at[pl.ds(sid * stripe, stripe), :])
        pltpu.sync_copy(seg_hbm.at[pl.ds(wid * epw, epw)], idx_v)
        plsc.subcore_barrier()

        def xsrc(c):
            return x_hbm.at[pl.ds(row_j * epw + c * ch, ch),
                            pl.ds(lane_a * units, units)]

        rows = (rows0, rows1)
        pltpu.async_copy(xsrc(0), rows0, sem_g)

        def drain(c):
            for jj in range(2):
                @pl.when(c % 2 == jj)
                def _():
                    for bq in range(nb):
                        ids = idx_v.at[pl.ds((c * nb + bq) * batch, batch)]
                        pltpu.make_async_copy(
                            rows[jj].at[pl.ds(bq * batch, batch), :],
                            acc_sh.at[ids], sem_s).wait()
                        pltpu.make_async_copy(ones_v, cnt_sh.at[ids],
                                              sem_s).wait()

        def chunk(c, carry):
            for jj in range(2):
                @pl.when(c % 2 == jj)
                def _():
                    buf = rows[jj]
                    pltpu.make_async_copy(xsrc(c), buf, sem_g).wait()

                    @pl.when(c >= 1)
                    def _():
                        drain(c - 1)

                    @pl.when(c + 1 < nchunk)
                    def _():
                        pltpu.async_copy(xsrc(c + 1), rows[1 - jj], sem_g)

                    for bq in range(nb):
                        ids = idx_v.at[pl.ds((c * nb + bq) * batch, batch)]
                        pltpu.async_copy(buf.at[pl.ds(bq * batch, batch), :],
                                         acc_sh.at[ids], sem_s, add=True)
                        pltpu.async_copy(ones_v, cnt_sh.at[ids], sem_s,
                                         add=True)
            return carry
        lax.fori_loop(0, nchunk, chunk, None)
        drain(nchunk - 1)
        plsc.subcore_barrier()

        pltpu.sync_copy(acc_sh.at[pl.ds(sid * stripe, stripe), :],
                        out_hbm.at[cid, pl.ds(sid * stripe, stripe), :])
        pltpu.sync_copy(cnt_sh.at[pl.ds(sid * stripe, stripe), :],
                        cnt_hbm.at[cid, pl.ds(sid * stripe, stripe), :])

    fn = pl.kernel(
        body,
        out_type=(jax.ShapeDtypeStruct((nc, n_pad, units), _F32),
                  jax.ShapeDtypeStruct((nc, n_pad, units), _F32)),
        mesh=mesh,
        scratch_types=[
            pltpu.VMEM((ch, units), _F32),
            pltpu.VMEM((ch, units), _F32),
            pltpu.VMEM((epw,), jnp.int32),
            pltpu.VMEM((stripe, units), _F32),
            pltpu.VMEM((batch, units), _F32),
            pltpu.VMEM_SHARED((n_pad, units), _F32),
            pltpu.VMEM_SHARED((n_pad, units), _F32),
            pltpu.SemaphoreType.DMA,
            pltpu.SemaphoreType.DMA,
        ],
        compiler_params=pltpu.CompilerParams(use_tc_tiling_on_sc=False),
    )
    return fn(packed, seg)


def _final_body(p_ref, c_ref, srcq_ref, w4_ref, b4_ref, out_ref, *, nq):
    sp = p_ref[0] + p_ref[1]
    cp = c_ref[0] + c_ref[1]
    mean = jnp.where(cp > 0, sp / jnp.maximum(cp, 1.0), 0.0)[0:nq]
    res = jnp.dot(srcq_ref[...], w4_ref[...], preferred_element_type=_F32)
    out_ref[...] = jnp.maximum(mean + res + b4_ref[...], 0.0)


def _finalize(partial_p, counts_p, srcq, w4, b4):
    nq = srcq.shape[0]
    body = functools.partial(_final_body, nq=nq)
    return pl.pallas_call(
        body,
        out_shape=jax.ShapeDtypeStruct((nq, w4.shape[1]), _F32),
    )(partial_p, counts_p, srcq, w4, b4)


def kernel(src, etype, dst, segment_ids, w, coefficients, w0, b):
    n, in_dim = src.shape
    e = dst.shape[0]
    nbasis, _, units = w.shape
    nrel = coefficients.shape[0]
    bw = nbasis * units

    coef_table = coefficients[:, :, 0, 0]
    cwide = jnp.repeat(coef_table, units, axis=1).astype(_BF16)
    wcat = jnp.transpose(w, (1, 0, 2)).reshape(in_dim, bw).astype(_BF16)
    fold = (jnp.arange(bw)[:, None] % units
            == jnp.arange(units)[None, :]).astype(_BF16)

    be = 3200
    packed = _messages(etype, dst, cwide, wcat, fold, be=be)

    batch, nb = 80, 5
    n_pad = ((n + 511) // 512) * 512
    partial, counts = _segment_partials(packed, segment_ids, n_pad,
                                        batch=batch, nb=nb)

    partial_p = partial.reshape(2, n_pad // 4, 4 * units)
    counts_p = counts.reshape(2, n_pad // 4, 4 * units)
    srcq = src.reshape(n // 4, 4 * in_dim)
    w4 = jnp.kron(jnp.eye(4, dtype=_F32), w0)
    b4 = jnp.tile(b, 4).reshape(1, 4 * units)
    out_p = _finalize(partial_p, counts_p, srcq, w4, b4)
    return out_p.reshape(n, units)


# speedup vs baseline: 16.5874x; 1.1029x over previous
import functools

import jax
import jax.numpy as jnp
from jax import lax
from jax.experimental import pallas as pl
from jax.experimental.pallas import tpu as pltpu
from jax.experimental.pallas import tpu_sc as plsc

_F32 = jnp.float32
_BF16 = jnp.bfloat16


def _message_body(et_ref, d0_ref, d1_ref, d2_ref, d3_ref,
                  cwide_ref, wcat_ref, fold_ref, out_ref,
                  *, be, g_off, g4, nrel, units):
    i = pl.program_id(0)
    xs = []
    for aa, d_ref in enumerate((d0_ref, d1_ref, d2_ref, d3_ref)):
        et = et_ref[pl.ds((g_off + aa * g4 + i) * be, be)][None, :]
        oh = (lax.broadcasted_iota(jnp.int32, (nrel, be), 0)
              == et).astype(_BF16)
        coefwide = lax.dot_general(oh, cwide_ref[...],
                                   (((0,), (0,)), ((), ())),
                                   preferred_element_type=_F32)
        h = jnp.dot(d_ref[...].astype(_BF16), wcat_ref[...],
                    preferred_element_type=_F32)
        xw = (coefwide * h).astype(_BF16)
        xs.append(jnp.dot(xw, fold_ref[...], preferred_element_type=_F32))
    out_ref[...] = jnp.concatenate(xs, axis=1)


def _messages(etype, dst, cwide, wcat, fold, *, be, g_off, g_cnt):
    e, in_dim = dst.shape
    g4 = g_cnt // 4
    nrel, bw = cwide.shape
    units = fold.shape[1]
    body = functools.partial(_message_body, be=be, g_off=g_off, g4=g4,
                             nrel=nrel, units=units)
    dspec = [pl.BlockSpec((be, in_dim),
                          lambda i, aa=aa, g4=g4: (g_off + aa * g4 + i, 0))
             for aa in range(4)]
    return pl.pallas_call(
        body,
        grid=(g4,),
        in_specs=[pl.BlockSpec((e,), lambda i: (0,))] + dspec + [
            pl.BlockSpec((nrel, bw), lambda i: (0, 0)),
            pl.BlockSpec((in_dim, bw), lambda i: (0, 0)),
            pl.BlockSpec((bw, units), lambda i: (0, 0)),
        ],
        out_specs=pl.BlockSpec((be, 4 * units), lambda i: (i, 0)),
        out_shape=jax.ShapeDtypeStruct((g_cnt * be // 4, 4 * units), _F32),
    )(etype, dst, dst, dst, dst, cwide, wcat, fold)


def _segment_partials(packed, seg, n_pad, *, batch, nb, e_off):
    e4, lanes = packed.shape
    e = e4 * 4
    units = lanes // 4
    nc, ns = 2, 16
    nw = nc * ns
    epw = e // nw
    ch = batch * nb
    nchunk = epw // ch
    stripe = n_pad // ns
    mesh = plsc.VectorSubcoreMesh(core_axis_name="c", subcore_axis_name="s")

    def body(x_hbm, seg_hbm, out_hbm, cnt_hbm,
             rows0, rows1, idx_v, zbuf, ones_v, acc_sh, cnt_sh, sem_g, sem_s):
        cid = lax.axis_index("c")
        sid = lax.axis_index("s")
        wid = sid * nc + cid
        lane_a = wid // 8
        row_j = wid % 8

        def zrow(r, carry):
            for k in range(units // 16):
                zbuf[r, pl.ds(k * 16, 16)] = jnp.zeros((16,), _F32)
            return carry
        lax.fori_loop(0, stripe, zrow, None)

        def orow(r, carry):
            for k in range(units // 16):
                ones_v[r, pl.ds(k * 16, 16)] = jnp.ones((16,), _F32)
            return carry
        lax.fori_loop(0, batch, orow, None)

        pltpu.sync_copy(zbuf, acc_sh.at[pl.ds(sid * stripe, stripe), :])
        pltpu.sync_copy(zbuf, cnt_sh.at[pl.ds(sid * stripe, stripe), :])
        pltpu.sync_copy(seg_hbm.at[pl.ds(e_off + wid * epw, epw)], idx_v)
        plsc.subcore_barrier()

        def xsrc(c):
            return x_hbm.at[pl.ds(row_j * epw + c * ch, ch),
                            pl.ds(lane_a * units, units)]

        rows = (rows0, rows1)
        pltpu.async_copy(xsrc(0), rows0, sem_g)

        def drain(c):
            for jj in range(2):
                @pl.when(c % 2 == jj)
                def _():
                    for bq in range(nb):
                        ids = idx_v.at[pl.ds((c * nb + bq) * batch, batch)]
                        pltpu.make_async_copy(
                            rows[jj].at[pl.ds(bq * batch, batch), :],
                            acc_sh.at[ids], sem_s).wait()
                        pltpu.make_async_copy(ones_v, cnt_sh.at[ids],
                                              sem_s).wait()

        def chunk(c, carry):
            for jj in range(2):
                @pl.when(c % 2 == jj)
                def _():
                    buf = rows[jj]
                    pltpu.make_async_copy(xsrc(c), buf, sem_g).wait()

                    @pl.when(c >= 1)
                    def _():
                        drain(c - 1)

                    @pl.when(c + 1 < nchunk)
                    def _():
                        pltpu.async_copy(xsrc(c + 1), rows[1 - jj], sem_g)

                    for bq in range(nb):
                        ids = idx_v.at[pl.ds((c * nb + bq) * batch, batch)]
                        pltpu.async_copy(buf.at[pl.ds(bq * batch, batch), :],
                                         acc_sh.at[ids], sem_s, add=True)
                        pltpu.async_copy(ones_v, cnt_sh.at[ids], sem_s,
                                         add=True)
            return carry
        lax.fori_loop(0, nchunk, chunk, None)
        drain(nchunk - 1)
        plsc.subcore_barrier()

        pltpu.sync_copy(acc_sh.at[pl.ds(sid * stripe, stripe), :],
                        out_hbm.at[cid, pl.ds(sid * stripe, stripe), :])
        pltpu.sync_copy(cnt_sh.at[pl.ds(sid * stripe, stripe), :],
                        cnt_hbm.at[cid, pl.ds(sid * stripe, stripe), :])

    fn = pl.kernel(
        body,
        out_type=(jax.ShapeDtypeStruct((nc, n_pad, units), _F32),
                  jax.ShapeDtypeStruct((nc, n_pad, units), _F32)),
        mesh=mesh,
        scratch_types=[
            pltpu.VMEM((ch, units), _F32),
            pltpu.VMEM((ch, units), _F32),
            pltpu.VMEM((epw,), jnp.int32),
            pltpu.VMEM((stripe, units), _F32),
            pltpu.VMEM((batch, units), _F32),
            pltpu.VMEM_SHARED((n_pad, units), _F32),
            pltpu.VMEM_SHARED((n_pad, units), _F32),
            pltpu.SemaphoreType.DMA,
            pltpu.SemaphoreType.DMA,
        ],
        compiler_params=pltpu.CompilerParams(use_tc_tiling_on_sc=False),
    )
    return fn(packed, seg)


def _final_body(pa_ref, pb_ref, ca_ref, cb_ref, srcq_ref, w4_ref, b4_ref,
                out_ref, *, nq):
    sp = pa_ref[0] + pa_ref[1] + pb_ref[0] + pb_ref[1]
    cp = ca_ref[0] + ca_ref[1] + cb_ref[0] + cb_ref[1]
    mean = jnp.where(cp > 0, sp / jnp.maximum(cp, 1.0), 0.0)[0:nq]
    res = jnp.dot(srcq_ref[...], w4_ref[...], preferred_element_type=_F32)
    out_ref[...] = jnp.maximum(mean + res + b4_ref[...], 0.0)


def _finalize(pa, pb, ca, cb, srcq, w4, b4):
    nq = srcq.shape[0]
    body = functools.partial(_final_body, nq=nq)
    return pl.pallas_call(
        body,
        out_shape=jax.ShapeDtypeStruct((nq, w4.shape[1]), _F32),
    )(pa, pb, ca, cb, srcq, w4, b4)


def kernel(src, etype, dst, segment_ids, w, coefficients, w0, b):
    n, in_dim = src.shape
    e = dst.shape[0]
    nbasis, _, units = w.shape
    nrel = coefficients.shape[0]
    bw = nbasis * units

    coef_table = coefficients[:, :, 0, 0]
    cwide = jnp.repeat(coef_table, units, axis=1).astype(_BF16)
    wcat = jnp.transpose(w, (1, 0, 2)).reshape(in_dim, bw).astype(_BF16)
    fold = (jnp.arange(bw)[:, None] % units
            == jnp.arange(units)[None, :]).astype(_BF16)

    be = 3200
    batch, nb = 80, 5
    n_pad = ((n + 511) // 512) * 512
    g = e // be
    g_a = (3 * g // 5 + 3) // 4 * 4
    e_a = g_a * be
    packed_a = _messages(etype, dst, cwide, wcat, fold,
                         be=be, g_off=0, g_cnt=g_a)
    pa, ca = _segment_partials(packed_a, segment_ids, n_pad,
                               batch=batch, nb=nb, e_off=0)
    packed_b = _messages(etype, dst, cwide, wcat, fold,
                         be=be, g_off=g_a, g_cnt=g - g_a)
    pb, cb = _segment_partials(packed_b, segment_ids, n_pad,
                               batch=batch, nb=nb, e_off=e_a)

    npq = n_pad // 4
    srcq = src.reshape(n // 4, 4 * in_dim)
    w4 = jnp.kron(jnp.eye(4, dtype=_F32), w0)
    b4 = jnp.tile(b, 4).reshape(1, 4 * units)
    out_p = _finalize(pa.reshape(2, npq, 4 * units),
                      pb.reshape(2, npq, 4 * units),
                      ca.reshape(2, npq, 4 * units),
                      cb.reshape(2, npq, 4 * units), srcq, w4, b4)
    return out_p.reshape(n, units)
